# baseline (speedup 1.0000x reference)
import functools

import jax
import jax.numpy as jnp
from jax.experimental import pallas as pl
from jax.experimental.pallas import tpu as pltpu

INPUT_DIM = 2048
OUTPUT_DIM = 2048
LORA_R = 64
NUM_EXPERTS = 8
LORA_ALPHA = 8.0
SCALING = LORA_ALPHA / LORA_R
ER = NUM_EXPERTS * LORA_R

BM = 256


def _body(xb, wg, bg, a2, b2, ob):
    xv = xb[...]
    logits = jnp.dot(xv, wg[...], preferred_element_type=jnp.float32) + bg[...]
    col = jax.lax.broadcasted_iota(jnp.int32, (BM, 128), 1)
    valid = col < NUM_EXPERTS
    lg = jnp.where(valid, logits, -jnp.inf)
    mx = jnp.max(lg, axis=1, keepdims=True)
    ex = jnp.exp(lg - mx)
    sm = ex / jnp.sum(ex, axis=1, keepdims=True)
    m1 = jnp.max(sm, axis=1, keepdims=True)
    i1 = jnp.min(jnp.where(sm == m1, col, 128), axis=1, keepdims=True)
    sm2 = jnp.where(col == i1, -1.0, sm)
    m2 = jnp.max(sm2, axis=1, keepdims=True)
    i2 = jnp.min(jnp.where(sm2 == m2, col, 128), axis=1, keepdims=True)
    denom = m1 + m2
    w1 = m1 / denom
    w2 = m2 / denom
    ecol = jax.lax.broadcasted_iota(jnp.int32, (BM, ER), 1) // LORA_R
    gates = jnp.where(ecol == i1, w1, 0.0) + jnp.where(ecol == i2, w2, 0.0)
    h = jnp.dot(xv, a2[...], preferred_element_type=jnp.float32)
    hw = h * gates
    ob[...] = jnp.dot(hw, b2[...], preferred_element_type=jnp.float32) * SCALING


@jax.jit
def _run(flat, wg, bg, a2, b2):
    n = flat.shape[0]
    grid = (n // BM,)
    return pl.pallas_call(
        _body,
        grid=grid,
        in_specs=[
            pl.BlockSpec((BM, INPUT_DIM), lambda i: (i, 0)),
            pl.BlockSpec((INPUT_DIM, 128), lambda i: (0, 0)),
            pl.BlockSpec((1, 128), lambda i: (0, 0)),
            pl.BlockSpec((INPUT_DIM, ER), lambda i: (0, 0)),
            pl.BlockSpec((ER, OUTPUT_DIM), lambda i: (0, 0)),
        ],
        out_specs=pl.BlockSpec((BM, OUTPUT_DIM), lambda i: (i, 0)),
        out_shape=jax.ShapeDtypeStruct((n, OUTPUT_DIM), jnp.float32),
        compiler_params=pltpu.CompilerParams(
            dimension_semantics=("arbitrary",),
        ),
    )(flat, wg, bg, a2, b2)


def kernel(x, W_gate, b_gate, A, B):
    flat = x.reshape(-1, x.shape[-1])
    wg = jnp.zeros((INPUT_DIM, 128), jnp.float32).at[:, :NUM_EXPERTS].set(W_gate.T)
    bg = jnp.zeros((1, 128), jnp.float32).at[0, :NUM_EXPERTS].set(b_gate)
    a2 = A.transpose(2, 0, 1).reshape(INPUT_DIM, ER)
    b2 = B.transpose(0, 2, 1).reshape(ER, OUTPUT_DIM)
    out = _run(flat, wg, bg, a2, b2)
    return out.reshape(x.shape[:-1] + (OUTPUT_DIM,))


# speedup vs baseline: 1.1727x; 1.1727x over previous
import functools

import jax
import jax.numpy as jnp
from jax.experimental import pallas as pl
from jax.experimental.pallas import tpu as pltpu

INPUT_DIM = 2048
OUTPUT_DIM = 2048
LORA_R = 64
NUM_EXPERTS = 8
LORA_ALPHA = 8.0
SCALING = LORA_ALPHA / LORA_R
ER = NUM_EXPERTS * LORA_R

BM = 1024


def _body(xb, wg, bg, a2, b2, ob):
    xv = xb[...]
    logits = jnp.dot(xv, wg[...], preferred_element_type=jnp.float32) + bg[...]
    col = jax.lax.broadcasted_iota(jnp.int32, (BM, 128), 1)
    valid = col < NUM_EXPERTS
    lg = jnp.where(valid, logits, -jnp.inf)
    mx = jnp.max(lg, axis=1, keepdims=True)
    ex = jnp.exp(lg - mx)
    sm = ex / jnp.sum(ex, axis=1, keepdims=True)
    m1 = jnp.max(sm, axis=1, keepdims=True)
    i1 = jnp.min(jnp.where(sm == m1, col, 128), axis=1, keepdims=True)
    sm2 = jnp.where(col == i1, -1.0, sm)
    m2 = jnp.max(sm2, axis=1, keepdims=True)
    i2 = jnp.min(jnp.where(sm2 == m2, col, 128), axis=1, keepdims=True)
    denom = m1 + m2
    w1 = m1 / denom
    w2 = m2 / denom
    ecol = jax.lax.broadcasted_iota(jnp.int32, (BM, ER), 1) // LORA_R
    gates = jnp.where(ecol == i1, w1, 0.0) + jnp.where(ecol == i2, w2, 0.0)
    h = jnp.dot(xv, a2[...], preferred_element_type=jnp.float32)
    hw = h * gates
    ob[...] = jnp.dot(hw, b2[...], preferred_element_type=jnp.float32) * SCALING


@jax.jit
def _run(flat, wg, bg, a2, b2):
    n = flat.shape[0]
    grid = (n // BM,)
    return pl.pallas_call(
        _body,
        grid=grid,
        in_specs=[
            pl.BlockSpec((BM, INPUT_DIM), lambda i: (i, 0)),
            pl.BlockSpec((INPUT_DIM, 128), lambda i: (0, 0)),
            pl.BlockSpec((1, 128), lambda i: (0, 0)),
            pl.BlockSpec((INPUT_DIM, ER), lambda i: (0, 0)),
            pl.BlockSpec((ER, OUTPUT_DIM), lambda i: (0, 0)),
        ],
        out_specs=pl.BlockSpec((BM, OUTPUT_DIM), lambda i: (i, 0)),
        out_shape=jax.ShapeDtypeStruct((n, OUTPUT_DIM), jnp.float32),
        compiler_params=pltpu.CompilerParams(
            dimension_semantics=("arbitrary",),
        ),
    )(flat, wg, bg, a2, b2)


def kernel(x, W_gate, b_gate, A, B):
    flat = x.reshape(-1, x.shape[-1])
    wg = jnp.zeros((INPUT_DIM, 128), jnp.float32).at[:, :NUM_EXPERTS].set(W_gate.T)
    bg = jnp.zeros((1, 128), jnp.float32).at[0, :NUM_EXPERTS].set(b_gate)
    a2 = A.transpose(2, 0, 1).reshape(INPUT_DIM, ER)
    b2 = B.transpose(0, 2, 1).reshape(ER, OUTPUT_DIM)
    out = _run(flat, wg, bg, a2, b2)
    return out.reshape(x.shape[:-1] + (OUTPUT_DIM,))
